# baseline (speedup 1.0000x reference)
import functools

import jax
import jax.numpy as jnp
from jax import lax
from jax.experimental import pallas as pl
from jax.experimental.pallas import tpu as pltpu
from jax.experimental.pallas import tpu_sc as plsc

_F32 = jnp.float32


def _fps_body(x_ref, y_ref, z_ref, ox_ref, oy_ref, oz_ref, *, npoint):
    x = x_ref[...]
    y = y_ref[...]
    z = z_ref[...]
    b, n = x.shape
    iota = lax.broadcasted_iota(jnp.int32, (b, n), 1)
    siota = lax.broadcasted_iota(jnp.int32, (b, npoint), 1)

    def body(s, carry):
        dist, cx, cy, cz, ax, ay, az = carry
        sel = siota == s
        ax = jnp.where(sel, cx, ax)
        ay = jnp.where(sel, cy, ay)
        az = jnp.where(sel, cz, az)
        d = (x - cx) ** 2 + (y - cy) ** 2 + (z - cz) ** 2
        dist = jnp.minimum(dist, d)
        mx = jnp.max(dist, axis=1, keepdims=True)
        am = jnp.min(jnp.where(dist >= mx, iota, n), axis=1, keepdims=True)
        oh = (iota == am).astype(_F32)
        ncx = jnp.sum(x * oh, axis=1, keepdims=True)
        ncy = jnp.sum(y * oh, axis=1, keepdims=True)
        ncz = jnp.sum(z * oh, axis=1, keepdims=True)
        return dist, ncx, ncy, ncz, ax, ay, az

    init = (
        jnp.full((b, n), 1e10, _F32),
        x[:, 0:1],
        y[:, 0:1],
        z[:, 0:1],
        jnp.zeros((b, npoint), _F32),
        jnp.zeros((b, npoint), _F32),
        jnp.zeros((b, npoint), _F32),
    )
    _, _, _, _, ax, ay, az = lax.fori_loop(0, npoint, body, init)
    ox_ref[...] = ax
    oy_ref[...] = ay
    oz_ref[...] = az


def _fps(x, y, z, npoint):
    b, n = x.shape
    return pl.pallas_call(
        functools.partial(_fps_body, npoint=npoint),
        out_shape=[jax.ShapeDtypeStruct((b, npoint), _F32)] * 3,
    )(x, y, z)


def _bq_body(xr_ref, yr_ref, zr_ref, cx_ref, cy_ref, cz_ref, o_ref, *, n, nb,
             cs, k, r2):
    xr = xr_ref[0][None]
    yr = yr_ref[0][None]
    zr = zr_ref[0][None]
    cx = cx_ref[...].reshape(cs, 1, 1)
    cy = cy_ref[...].reshape(cs, 1, 1)
    cz = cz_ref[...].reshape(cs, 1, 1)
    dx = cx - xr
    dy = cy - yr
    dz = cz - zr
    d = dx * dx + dy * dy + dz * dz
    maskf = (d <= r2).astype(_F32)
    u = (lax.broadcasted_iota(jnp.int32, (128, 128), 0)
         <= lax.broadcasted_iota(jnp.int32, (128, 128), 1)).astype(_F32)
    within = jnp.dot(maskf.reshape(cs * nb, 128), u,
                     preferred_element_type=_F32).reshape(cs, nb, 128)
    bsum = within[:, :, 127:128].reshape(cs, nb)
    us = (lax.broadcasted_iota(jnp.int32, (nb, nb), 0)
          < lax.broadcasted_iota(jnp.int32, (nb, nb), 1)).astype(_F32)
    offs = jnp.dot(bsum, us, preferred_element_type=_F32)
    m = jnp.sum(bsum, axis=1, keepdims=True)
    cnt = (within + offs[:, :, None]).reshape(cs, n)
    kk = lax.broadcasted_iota(jnp.int32, (cs, k), 1).astype(_F32)
    keff = jnp.where(kk + 1.0 <= m, kk, 0.0)
    cmp = (cnt[:, None, :] <= keff[:, :, None]).astype(_F32)
    idxf = jnp.sum(cmp, axis=2)
    b_id = pl.program_id(0)
    o_ref[...] = idxf.astype(jnp.int32) + b_id * n


def _ball_query(x, y, z, cx, cy, cz, radius, k, cs):
    b, n = x.shape
    s = cx.shape[1]
    nb = n // 128
    xr = x.reshape(b, nb, 128)
    yr = y.reshape(b, nb, 128)
    zr = z.reshape(b, nb, 128)
    nch = s // cs
    grid = (b, nch)
    cx3 = cx.reshape(b * nch, 1, cs)
    cy3 = cy.reshape(b * nch, 1, cs)
    cz3 = cz.reshape(b * nch, 1, cs)
    return pl.pallas_call(
        functools.partial(_bq_body, n=n, nb=nb, cs=cs, k=k, r2=radius * radius),
        grid=grid,
        in_specs=[
            pl.BlockSpec((1, nb, 128), lambda i, j: (i, 0, 0)),
            pl.BlockSpec((1, nb, 128), lambda i, j: (i, 0, 0)),
            pl.BlockSpec((1, nb, 128), lambda i, j: (i, 0, 0)),
            pl.BlockSpec((1, 1, cs), lambda i, j: (i * nch + j, 0, 0)),
            pl.BlockSpec((1, 1, cs), lambda i, j: (i * nch + j, 0, 0)),
            pl.BlockSpec((1, 1, cs), lambda i, j: (i * nch + j, 0, 0)),
        ],
        out_specs=pl.BlockSpec((cs, k), lambda i, j: (i * nch + j, 0)),
        out_shape=jax.ShapeDtypeStruct((b * s, k), jnp.int32),
    )(xr, yr, zr, cx3, cy3, cz3)


def _sc_gather(table, gidx, d_pad):
    r = gidx.shape[0]
    nw = 32
    rpw = r // nw
    ch = 8
    while ch * 2 <= rpw and ch * 2 * d_pad * 4 <= 300 * 1024:
        ch *= 2
    nchunks = rpw // ch
    mesh = plsc.VectorSubcoreMesh(core_axis_name="c", subcore_axis_name="s")

    @functools.partial(
        pl.kernel,
        mesh=mesh,
        out_type=jax.ShapeDtypeStruct((r, d_pad), _F32),
        scratch_types=[
            pltpu.VMEM((ch,), jnp.int32),
            pltpu.VMEM((ch, d_pad), _F32),
            pltpu.SemaphoreType.DMA,
        ],
        compiler_params=pltpu.CompilerParams(use_tc_tiling_on_sc=False),
    )
    def gather_k(table_hbm, idx_hbm, out_hbm, idx_v, rows_v, sem):
        wid = lax.axis_index("s") * 2 + lax.axis_index("c")
        base = wid * rpw

        def body(c, carry):
            off = base + c * ch
            pltpu.sync_copy(idx_hbm.at[pl.ds(off, ch)], idx_v)
            pltpu.async_copy(table_hbm.at[idx_v], rows_v, sem).wait()
            pltpu.sync_copy(rows_v, out_hbm.at[pl.ds(off, ch)])
            return carry

        lax.fori_loop(0, nchunks, body, 0)

    return gather_k(table, gidx)


def _accum_stats(y, s_ref, q_ref):
    @pl.when(pl.program_id(0) == 0)
    def _():
        s_ref[...] = jnp.zeros_like(s_ref)
        q_ref[...] = jnp.zeros_like(q_ref)

    s_ref[...] += jnp.sum(y, axis=0, keepdims=True)
    q_ref[...] += jnp.sum(y * y, axis=0, keepdims=True)


def _norm_consts(s_ref, q_ref, g_ref, bt_ref, nrows):
    mu = s_ref[...] * (1.0 / nrows)
    var = jnp.maximum(q_ref[...] * (1.0 / nrows) - mu * mu, 0.0)
    sc = lax.rsqrt(var + 1e-5) * g_ref[...]
    return mu, sc, bt_ref[...]


def _group_lin_body(g_ref, cx_ref, cy_ref, cz_ref, wt_ref, bb_ref, y_ref,
                    s_ref, q_ref, *, kk, radius):
    g = g_ref[...]
    crg = g.shape[0] // kk
    dpad = g.shape[1]
    g3 = g.reshape(crg, kk, dpad)
    lane = lax.broadcasted_iota(jnp.int32, (crg, 1, dpad), 2)
    cx = cx_ref[...].reshape(crg, 1, 1)
    cy = cy_ref[...].reshape(crg, 1, 1)
    cz = cz_ref[...].reshape(crg, 1, 1)
    off = jnp.where(lane == 0, -cx,
                    jnp.where(lane == 1, -cy,
                              jnp.where(lane == 2, -cz, 0.0)))
    dv = jnp.where(lane < 3, radius, 1.0)
    xin = ((g3 + off) / dv).reshape(crg * kk, dpad)
    y = jnp.dot(xin, wt_ref[...], preferred_element_type=_F32) + bb_ref[...]
    y_ref[...] = y
    _accum_stats(y, s_ref, q_ref)


def _lin_body(x_ref, wt_ref, bb_ref, y_ref, s_ref, q_ref):
    y = jnp.dot(x_ref[...], wt_ref[...],
                preferred_element_type=_F32) + bb_ref[...]
    y_ref[...] = y
    _accum_stats(y, s_ref, q_ref)


def _normlin_body(y_ref, si_ref, qi_ref, g_ref, bt_ref, wt_ref, bb_ref,
                  o_ref, s_ref, q_ref, *, nrows):
    mu, sc, bt = _norm_consts(si_ref, qi_ref, g_ref, bt_ref, nrows)
    x = jnp.maximum((y_ref[...] - mu) * sc + bt, 0.0)
    y = jnp.dot(x, wt_ref[...], preferred_element_type=_F32) + bb_ref[...]
    o_ref[...] = y
    _accum_stats(y, s_ref, q_ref)


def _pool_body(y_ref, si_ref, qi_ref, g_ref, bt_ref, o_ref, *, nrows, kk):
    mu, sc, bt = _norm_consts(si_ref, qi_ref, g_ref, bt_ref, nrows)
    x = jnp.maximum((y_ref[...] - mu) * sc + bt, 0.0)
    rg = x.shape[0] // kk
    o_ref[...] = jnp.max(x.reshape(rg, kk, x.shape[1]), axis=1)


def _finalize_body(y_ref, si_ref, qi_ref, g_ref, bt_ref, o_ref, *, nrows):
    mu, sc, bt = _norm_consts(si_ref, qi_ref, g_ref, bt_ref, nrows)
    o_ref[...] = jnp.maximum((y_ref[...] - mu) * sc + bt, 0.0)


def _head_body(y_ref, si_ref, qi_ref, g_ref, bt_ref, wt_ref, bb_ref, o_ref,
               *, nrows):
    mu, sc, bt = _norm_consts(si_ref, qi_ref, g_ref, bt_ref, nrows)
    x = jnp.maximum((y_ref[...] - mu) * sc + bt, 0.0)
    z = jnp.dot(x, wt_ref[...], preferred_element_type=_F32) + bb_ref[...]
    o_ref[...] = jax.nn.sigmoid(z)


def _prep_w(w, cin_pad):
    cout, cin = w.shape
    wt = jnp.zeros((cin_pad, cout), _F32)
    return wt.at[:cin, :].set(w.T)


def _chunk_rows(rows, cmax):
    c = 8192 if cmax <= 256 else 4096
    return min(rows, c)


def _vec_spec(cout):
    return pl.BlockSpec((1, cout), lambda i: (0, 0))


def _lin_call(x, w, bias, *, group=None):
    rows, cin_pad = x.shape
    cout = w.shape[0]
    cr = _chunk_rows(rows, max(cin_pad, cout))
    grid = (rows // cr,)
    wt = _prep_w(w, cin_pad)
    bb = bias.reshape(1, cout)
    out_shape = [
        jax.ShapeDtypeStruct((rows, cout), _F32),
        jax.ShapeDtypeStruct((1, cout), _F32),
        jax.ShapeDtypeStruct((1, cout), _F32),
    ]
    out_specs = [
        pl.BlockSpec((cr, cout), lambda i: (i, 0)),
        _vec_spec(cout),
        _vec_spec(cout),
    ]
    if group is None:
        return pl.pallas_call(
            _lin_body,
            grid=grid,
            in_specs=[
                pl.BlockSpec((cr, cin_pad), lambda i: (i, 0)),
                pl.BlockSpec((cin_pad, cout), lambda i: (0, 0)),
                _vec_spec(cout),
            ],
            out_specs=out_specs,
            out_shape=out_shape,
        )(x, wt, bb)
    cx, cy, cz, kk, radius = group
    crg = cr // kk
    return pl.pallas_call(
        functools.partial(_group_lin_body, kk=kk, radius=radius),
        grid=grid,
        in_specs=[
            pl.BlockSpec((cr, cin_pad), lambda i: (i, 0)),
            pl.BlockSpec((crg, 1), lambda i: (i, 0)),
            pl.BlockSpec((crg, 1), lambda i: (i, 0)),
            pl.BlockSpec((crg, 1), lambda i: (i, 0)),
            pl.BlockSpec((cin_pad, cout), lambda i: (0, 0)),
            _vec_spec(cout),
        ],
        out_specs=out_specs,
        out_shape=out_shape,
    )(x, cx, cy, cz, wt, bb)


def _normlin_call(y, s, q, g, bt, w, bias):
    rows, cin = y.shape
    cout = w.shape[0]
    cr = _chunk_rows(rows, max(cin, cout))
    grid = (rows // cr,)
    wt = _prep_w(w, cin)
    bb = bias.reshape(1, cout)
    return pl.pallas_call(
        functools.partial(_normlin_body, nrows=rows),
        grid=grid,
        in_specs=[
            pl.BlockSpec((cr, cin), lambda i: (i, 0)),
            _vec_spec(cin),
            _vec_spec(cin),
            _vec_spec(cin),
            _vec_spec(cin),
            pl.BlockSpec((cin, cout), lambda i: (0, 0)),
            _vec_spec(cout),
        ],
        out_specs=[
            pl.BlockSpec((cr, cout), lambda i: (i, 0)),
            _vec_spec(cout),
            _vec_spec(cout),
        ],
        out_shape=[
            jax.ShapeDtypeStruct((rows, cout), _F32),
            jax.ShapeDtypeStruct((1, cout), _F32),
            jax.ShapeDtypeStruct((1, cout), _F32),
        ],
    )(y, s, q, g.reshape(1, cin), bt.reshape(1, cin), wt, bb)


def _pool_call(y, s, q, g, bt, kk):
    rows, cin = y.shape
    cr = _chunk_rows(rows, cin)
    grid = (rows // cr,)
    return pl.pallas_call(
        functools.partial(_pool_body, nrows=rows, kk=kk),
        grid=grid,
        in_specs=[
            pl.BlockSpec((cr, cin), lambda i: (i, 0)),
            _vec_spec(cin),
            _vec_spec(cin),
            _vec_spec(cin),
            _vec_spec(cin),
        ],
        out_specs=pl.BlockSpec((cr // kk, cin), lambda i: (i, 0)),
        out_shape=jax.ShapeDtypeStruct((rows // kk, cin), _F32),
    )(y, s, q, g.reshape(1, cin), bt.reshape(1, cin))


def _finalize_call(y, s, q, g, bt):
    rows, cin = y.shape
    cr = _chunk_rows(rows, cin)
    grid = (rows // cr,)
    return pl.pallas_call(
        functools.partial(_finalize_body, nrows=rows),
        grid=grid,
        in_specs=[
            pl.BlockSpec((cr, cin), lambda i: (i, 0)),
            _vec_spec(cin),
            _vec_spec(cin),
            _vec_spec(cin),
            _vec_spec(cin),
        ],
        out_specs=pl.BlockSpec((cr, cin), lambda i: (i, 0)),
        out_shape=jax.ShapeDtypeStruct((rows, cin), _F32),
    )(y, s, q, g.reshape(1, cin), bt.reshape(1, cin))


def _head_call(y, s, q, g, bt, w2, b2):
    rows, cin = y.shape
    cout = w2.shape[0]
    cr = _chunk_rows(rows, cin)
    grid = (rows // cr,)
    wt = _prep_w(w2, cin)
    bb = b2.reshape(1, cout)
    return pl.pallas_call(
        functools.partial(_head_body, nrows=rows),
        grid=grid,
        in_specs=[
            pl.BlockSpec((cr, cin), lambda i: (i, 0)),
            _vec_spec(cin),
            _vec_spec(cin),
            _vec_spec(cin),
            _vec_spec(cin),
            pl.BlockSpec((cin, cout), lambda i: (0, 0)),
            _vec_spec(cout),
        ],
        out_specs=pl.BlockSpec((cr, cout), lambda i: (i, 0)),
        out_shape=jax.ShapeDtypeStruct((rows, cout), _F32),
    )(y, s, q, g.reshape(1, cin), bt.reshape(1, cin), wt, bb)


def _fp_body(x1_ref, y1_ref, z1_ref, x2_ref, y2_ref, z2_ref, p2_ref, o_ref,
             *, n1, n2):
    x1 = x1_ref[...].reshape(n1, 1)
    y1 = y1_ref[...].reshape(n1, 1)
    z1 = z1_ref[...].reshape(n1, 1)
    x2 = x2_ref[...].reshape(1, n2)
    y2 = y2_ref[...].reshape(1, n2)
    z2 = z2_ref[...].reshape(1, n2)
    dx = x1 - x2
    dy = y1 - y2
    dz = z1 - z2
    d = dx * dx + dy * dy + dz * dz
    iota = lax.broadcasted_iota(jnp.int32, (n1, n2), 1)
    picks = []
    for _ in range(3):
        mv = jnp.min(d, axis=1, keepdims=True)
        am = jnp.min(jnp.where(d <= mv, iota, n2), axis=1, keepdims=True)
        picks.append((1.0 / (mv + 1e-8), am))
        d = jnp.where(iota == am, jnp.inf, d)
    norm = picks[0][0] + picks[1][0] + picks[2][0]
    wm = jnp.zeros((n1, n2), _F32)
    for rec, am in picks:
        wm = wm + (iota == am).astype(_F32) * (rec / norm)
    o_ref[...] = jnp.dot(wm, p2_ref[0], preferred_element_type=_F32)


def _fp_interp(x1, y1, z1, x2, y2, z2, p2):
    b, n1 = x1.shape
    n2 = x2.shape[1]
    c2 = p2.shape[2]
    return pl.pallas_call(
        functools.partial(_fp_body, n1=n1, n2=n2),
        grid=(b,),
        in_specs=[
            pl.BlockSpec((1, 1, n1), lambda i: (i, 0, 0)),
            pl.BlockSpec((1, 1, n1), lambda i: (i, 0, 0)),
            pl.BlockSpec((1, 1, n1), lambda i: (i, 0, 0)),
            pl.BlockSpec((1, 1, n2), lambda i: (i, 0, 0)),
            pl.BlockSpec((1, 1, n2), lambda i: (i, 0, 0)),
            pl.BlockSpec((1, 1, n2), lambda i: (i, 0, 0)),
            pl.BlockSpec((1, n2, c2), lambda i: (i, 0, 0)),
        ],
        out_specs=pl.BlockSpec((n1, c2), lambda i: (i, 0)),
        out_shape=jax.ShapeDtypeStruct((b * n1, c2), _F32),
    )(x1.reshape(b, 1, n1), y1.reshape(b, 1, n1), z1.reshape(b, 1, n1),
      x2.reshape(b, 1, n2), y2.reshape(b, 1, n2), z2.reshape(b, 1, n2), p2)


def _pad_cols(a, cpad):
    rows, c = a.shape
    if c == cpad:
        return a
    return jnp.concatenate([a, jnp.zeros((rows, cpad - c), _F32)], axis=1)


def _set_abstraction(x, y, z, feats, npoint, radius, nsample, layers, cs):
    b, n = x.shape
    cfeat = feats.shape[2]
    cx, cy, cz = _fps(x, y, z, npoint)
    gidx = _ball_query(x, y, z, cx, cy, cz, radius, nsample, cs)
    cin = 3 + cfeat
    cpad = ((cin + 15) // 16) * 16
    table = jnp.concatenate(
        [x[..., None], y[..., None], z[..., None], feats], axis=2)
    table = _pad_cols(table.reshape(b * n, cin), cpad)
    rows = _sc_gather(table, gidx.reshape(-1), cpad)
    cenx = cx.reshape(b * npoint, 1)
    ceny = cy.reshape(b * npoint, 1)
    cenz = cz.reshape(b * npoint, 1)
    yv, s, q = _lin_call(rows, layers[0][0], layers[0][1],
                         group=(cenx, ceny, cenz, nsample, radius))
    g, bt = layers[0][2], layers[0][3]
    for (w, bias, g2, bt2) in layers[1:]:
        yv, s2, q2 = _normlin_call(yv, s, q, g, bt, w, bias)
        s, q, g, bt = s2, q2, g2, bt2
    pooled = _pool_call(yv, s, q, g, bt, nsample)
    return cx, cy, cz, pooled


def _feature_prop(x1, y1, z1, x2, y2, z2, p1flat, p2, layers, finalize):
    b, n1 = x1.shape
    interp = _fp_interp(x1, y1, z1, x2, y2, z2, p2)
    xin = jnp.concatenate([p1flat, interp], axis=1)
    yv, s, q = _lin_call(xin, layers[0][0], layers[0][1])
    g, bt = layers[0][2], layers[0][3]
    for (w, bias, g2, bt2) in layers[1:]:
        yv, s2, q2 = _normlin_call(yv, s, q, g, bt, w, bias)
        s, q, g, bt = s2, q2, g2, bt2
    if finalize:
        return _finalize_call(yv, s, q, g, bt)
    return yv, s, q, g, bt


def kernel(xyz, features, params):
    b, n, _ = xyz.shape
    x0 = xyz[..., 0]
    y0 = xyz[..., 1]
    z0 = xyz[..., 2]

    cx1, cy1, cz1, p1 = _set_abstraction(
        x0, y0, z0, features, 256, 0.05, 32, params['sa1'], cs=32)
    p1r = p1.reshape(b, 256, p1.shape[1])
    cx2, cy2, cz2, p2 = _set_abstraction(
        cx1, cy1, cz1, p1r, 128, 0.1, 64, params['sa2'], cs=128)
    p2r = p2.reshape(b, 128, p2.shape[1])
    cx3, cy3, cz3, p3 = _set_abstraction(
        cx2, cy2, cz2, p2r, 64, 0.2, 128, params['sa3'], cs=64)
    p3r = p3.reshape(b, 64, p3.shape[1])

    l2f = _feature_prop(cx2, cy2, cz2, cx3, cy3, cz3, p2, p3r,
                        params['fp3'], finalize=True)
    l2fr = l2f.reshape(b, 128, l2f.shape[1])
    l1f = _feature_prop(cx1, cy1, cz1, cx2, cy2, cz2, p1, l2fr,
                        params['fp2'], finalize=True)
    l1fr = l1f.reshape(b, 256, l1f.shape[1])
    f0flat = features.reshape(b * n, features.shape[2])
    yv, s, q, g, bt = _feature_prop(x0, y0, z0, cx1, cy1, cz1, f0flat, l1fr,
                                    params['fp1'], finalize=False)

    w1, b1, g1, bt1 = params['head1']
    yh, sh, qh = _normlin_call(yv, s, q, g, bt, w1, b1)
    w2, b2 = params['head2']
    out = _head_call(yh, sh, qh, g1, bt1, w2, b2)
    return out.reshape(b, n)


# speedup vs baseline: 1.1422x; 1.1422x over previous
import functools

import jax
import jax.numpy as jnp
from jax import lax
from jax.experimental import pallas as pl
from jax.experimental.pallas import tpu as pltpu
from jax.experimental.pallas import tpu_sc as plsc

_F32 = jnp.float32


def _fps_body(x_ref, y_ref, z_ref, ox_ref, oy_ref, oz_ref, *, npoint):
    x = x_ref[...]
    y = y_ref[...]
    z = z_ref[...]
    b, n = x.shape
    iota = lax.broadcasted_iota(jnp.int32, (b, n), 1)
    siota = lax.broadcasted_iota(jnp.int32, (b, npoint), 1)

    def body(s, carry):
        dist, cx, cy, cz, ax, ay, az = carry
        sel = siota == s
        ax = jnp.where(sel, cx, ax)
        ay = jnp.where(sel, cy, ay)
        az = jnp.where(sel, cz, az)
        d = (x - cx) ** 2 + (y - cy) ** 2 + (z - cz) ** 2
        dist = jnp.minimum(dist, d)
        mx = jnp.max(dist, axis=1, keepdims=True)
        am = jnp.min(jnp.where(dist >= mx, iota, n), axis=1, keepdims=True)
        oh = (iota == am).astype(_F32)
        ncx = jnp.sum(x * oh, axis=1, keepdims=True)
        ncy = jnp.sum(y * oh, axis=1, keepdims=True)
        ncz = jnp.sum(z * oh, axis=1, keepdims=True)
        return dist, ncx, ncy, ncz, ax, ay, az

    init = (
        jnp.full((b, n), 1e10, _F32),
        x[:, 0:1],
        y[:, 0:1],
        z[:, 0:1],
        jnp.zeros((b, npoint), _F32),
        jnp.zeros((b, npoint), _F32),
        jnp.zeros((b, npoint), _F32),
    )
    _, _, _, _, ax, ay, az = lax.fori_loop(0, npoint, body, init)
    ox_ref[...] = ax
    oy_ref[...] = ay
    oz_ref[...] = az


def _fps(x, y, z, npoint):
    b, n = x.shape
    return pl.pallas_call(
        functools.partial(_fps_body, npoint=npoint),
        out_shape=[jax.ShapeDtypeStruct((b, npoint), _F32)] * 3,
    )(x, y, z)


def _bq_body(xr_ref, yr_ref, zr_ref, cx_ref, cy_ref, cz_ref, o_ref, *, n, nb,
             cs, k, r2):
    xr = xr_ref[0][None]
    yr = yr_ref[0][None]
    zr = zr_ref[0][None]
    cx = cx_ref[...].reshape(cs, 1, 1)
    cy = cy_ref[...].reshape(cs, 1, 1)
    cz = cz_ref[...].reshape(cs, 1, 1)
    dx = cx - xr
    dy = cy - yr
    dz = cz - zr
    d = dx * dx + dy * dy + dz * dz
    maskf = (d <= r2).astype(_F32)
    u = (lax.broadcasted_iota(jnp.int32, (128, 128), 0)
         <= lax.broadcasted_iota(jnp.int32, (128, 128), 1)).astype(_F32)
    within = jnp.dot(maskf.reshape(cs * nb, 128), u,
                     preferred_element_type=_F32).reshape(cs, nb, 128)
    bsum = within[:, :, 127:128].reshape(cs, nb)
    us = (lax.broadcasted_iota(jnp.int32, (nb, nb), 0)
          < lax.broadcasted_iota(jnp.int32, (nb, nb), 1)).astype(_F32)
    offs = jnp.dot(bsum, us, preferred_element_type=_F32)
    m = jnp.sum(bsum, axis=1, keepdims=True)
    cnt = (within + offs[:, :, None]).reshape(cs, n)
    kk = lax.broadcasted_iota(jnp.int32, (cs, k), 1).astype(_F32)
    keff = jnp.where(kk + 1.0 <= m, kk, 0.0)
    cmp = (cnt[:, None, :] <= keff[:, :, None]).astype(_F32)
    idxf = jnp.sum(cmp, axis=2)
    b_id = pl.program_id(0)
    o_ref[...] = idxf.astype(jnp.int32) + b_id * n


def _ball_query(x, y, z, cx, cy, cz, radius, k, cs):
    b, n = x.shape
    s = cx.shape[1]
    nb = n // 128
    xr = x.reshape(b, nb, 128)
    yr = y.reshape(b, nb, 128)
    zr = z.reshape(b, nb, 128)
    nch = s // cs
    grid = (b, nch)
    cx3 = cx.reshape(b * nch, 1, cs)
    cy3 = cy.reshape(b * nch, 1, cs)
    cz3 = cz.reshape(b * nch, 1, cs)
    return pl.pallas_call(
        functools.partial(_bq_body, n=n, nb=nb, cs=cs, k=k, r2=radius * radius),
        grid=grid,
        in_specs=[
            pl.BlockSpec((1, nb, 128), lambda i, j: (i, 0, 0)),
            pl.BlockSpec((1, nb, 128), lambda i, j: (i, 0, 0)),
            pl.BlockSpec((1, nb, 128), lambda i, j: (i, 0, 0)),
            pl.BlockSpec((1, 1, cs), lambda i, j: (i * nch + j, 0, 0)),
            pl.BlockSpec((1, 1, cs), lambda i, j: (i * nch + j, 0, 0)),
            pl.BlockSpec((1, 1, cs), lambda i, j: (i * nch + j, 0, 0)),
        ],
        out_specs=pl.BlockSpec((cs, k), lambda i, j: (i * nch + j, 0)),
        out_shape=jax.ShapeDtypeStruct((b * s, k), jnp.int32),
    )(xr, yr, zr, cx3, cy3, cz3)


def _sc_gather(table, gidx, d_pad):
    r = gidx.shape[0]
    nrows_t = table.shape[0]
    nw = 32
    rpw = r // nw
    ch = 8
    while ch * 2 <= rpw and ch * 2 * d_pad * 4 <= 300 * 1024:
        ch *= 2
    nchunks = rpw // ch
    mesh = plsc.VectorSubcoreMesh(core_axis_name="c", subcore_axis_name="s")

    @functools.partial(
        pl.kernel,
        mesh=mesh,
        out_type=jax.ShapeDtypeStruct((r, d_pad), _F32),
        scratch_types=[
            pltpu.VMEM((ch,), jnp.int32),
            pltpu.VMEM((ch, d_pad), _F32),
            pltpu.VMEM_SHARED((nrows_t, d_pad), _F32),
            pltpu.SemaphoreType.DMA,
        ],
        compiler_params=pltpu.CompilerParams(use_tc_tiling_on_sc=False),
    )
    def gather_k(table_hbm, idx_hbm, out_hbm, idx_v, rows_v, table_sh, sem):
        sid = lax.axis_index("s")
        wid = sid * 2 + lax.axis_index("c")
        base = wid * rpw

        @pl.when(sid == 0)
        def _():
            pltpu.sync_copy(table_hbm, table_sh)

        plsc.subcore_barrier()

        def body(c, carry):
            off = base + c * ch
            pltpu.sync_copy(idx_hbm.at[pl.ds(off, ch)], idx_v)
            pltpu.async_copy(table_sh.at[idx_v], rows_v, sem).wait()
            pltpu.sync_copy(rows_v, out_hbm.at[pl.ds(off, ch)])
            return carry

        lax.fori_loop(0, nchunks, body, 0)

    return gather_k(table, gidx)


def _accum_stats(y, s_ref, q_ref):
    @pl.when(pl.program_id(0) == 0)
    def _():
        s_ref[...] = jnp.zeros_like(s_ref)
        q_ref[...] = jnp.zeros_like(q_ref)

    s_ref[...] += jnp.sum(y, axis=0, keepdims=True)
    q_ref[...] += jnp.sum(y * y, axis=0, keepdims=True)


def _norm_consts(s_ref, q_ref, g_ref, bt_ref, nrows):
    mu = s_ref[...] * (1.0 / nrows)
    var = jnp.maximum(q_ref[...] * (1.0 / nrows) - mu * mu, 0.0)
    sc = lax.rsqrt(var + 1e-5) * g_ref[...]
    return mu, sc, bt_ref[...]


def _group_lin_body(g_ref, cx_ref, cy_ref, cz_ref, wt_ref, bb_ref, y_ref,
                    s_ref, q_ref, *, kk, radius):
    g = g_ref[...]
    crg = g.shape[0] // kk
    dpad = g.shape[1]
    g3 = g.reshape(crg, kk, dpad)
    lane = lax.broadcasted_iota(jnp.int32, (crg, 1, dpad), 2)
    cx = cx_ref[...].reshape(crg, 1, 1)
    cy = cy_ref[...].reshape(crg, 1, 1)
    cz = cz_ref[...].reshape(crg, 1, 1)
    off = jnp.where(lane == 0, -cx,
                    jnp.where(lane == 1, -cy,
                              jnp.where(lane == 2, -cz, 0.0)))
    dv = jnp.where(lane < 3, radius, 1.0)
    xin = ((g3 + off) / dv).reshape(crg * kk, dpad)
    y = jnp.dot(xin, wt_ref[...], preferred_element_type=_F32) + bb_ref[...]
    y_ref[...] = y
    _accum_stats(y, s_ref, q_ref)


def _lin_body(x_ref, wt_ref, bb_ref, y_ref, s_ref, q_ref):
    y = jnp.dot(x_ref[...], wt_ref[...],
                preferred_element_type=_F32) + bb_ref[...]
    y_ref[...] = y
    _accum_stats(y, s_ref, q_ref)


def _normlin_body(y_ref, si_ref, qi_ref, g_ref, bt_ref, wt_ref, bb_ref,
                  o_ref, s_ref, q_ref, *, nrows):
    mu, sc, bt = _norm_consts(si_ref, qi_ref, g_ref, bt_ref, nrows)
    x = jnp.maximum((y_ref[...] - mu) * sc + bt, 0.0)
    y = jnp.dot(x, wt_ref[...], preferred_element_type=_F32) + bb_ref[...]
    o_ref[...] = y
    _accum_stats(y, s_ref, q_ref)


def _pool_body(y_ref, si_ref, qi_ref, g_ref, bt_ref, o_ref, *, nrows, kk):
    mu, sc, bt = _norm_consts(si_ref, qi_ref, g_ref, bt_ref, nrows)
    x = jnp.maximum((y_ref[...] - mu) * sc + bt, 0.0)
    rg = x.shape[0] // kk
    o_ref[...] = jnp.max(x.reshape(rg, kk, x.shape[1]), axis=1)


def _finalize_body(y_ref, si_ref, qi_ref, g_ref, bt_ref, o_ref, *, nrows):
    mu, sc, bt = _norm_consts(si_ref, qi_ref, g_ref, bt_ref, nrows)
    o_ref[...] = jnp.maximum((y_ref[...] - mu) * sc + bt, 0.0)


def _head_body(y_ref, si_ref, qi_ref, g_ref, bt_ref, wt_ref, bb_ref, o_ref,
               *, nrows):
    mu, sc, bt = _norm_consts(si_ref, qi_ref, g_ref, bt_ref, nrows)
    x = jnp.maximum((y_ref[...] - mu) * sc + bt, 0.0)
    z = jnp.dot(x, wt_ref[...], preferred_element_type=_F32) + bb_ref[...]
    o_ref[...] = jax.nn.sigmoid(z)


def _prep_w(w, cin_pad):
    cout, cin = w.shape
    wt = jnp.zeros((cin_pad, cout), _F32)
    return wt.at[:cin, :].set(w.T)


def _chunk_rows(rows, cmax):
    c = 8192 if cmax <= 256 else 4096
    return min(rows, c)


def _vec_spec(cout):
    return pl.BlockSpec((1, cout), lambda i: (0, 0))


def _lin_call(x, w, bias, *, group=None):
    rows, cin_pad = x.shape
    cout = w.shape[0]
    cr = _chunk_rows(rows, max(cin_pad, cout))
    grid = (rows // cr,)
    wt = _prep_w(w, cin_pad)
    bb = bias.reshape(1, cout)
    out_shape = [
        jax.ShapeDtypeStruct((rows, cout), _F32),
        jax.ShapeDtypeStruct((1, cout), _F32),
        jax.ShapeDtypeStruct((1, cout), _F32),
    ]
    out_specs = [
        pl.BlockSpec((cr, cout), lambda i: (i, 0)),
        _vec_spec(cout),
        _vec_spec(cout),
    ]
    if group is None:
        return pl.pallas_call(
            _lin_body,
            grid=grid,
            in_specs=[
                pl.BlockSpec((cr, cin_pad), lambda i: (i, 0)),
                pl.BlockSpec((cin_pad, cout), lambda i: (0, 0)),
                _vec_spec(cout),
            ],
            out_specs=out_specs,
            out_shape=out_shape,
        )(x, wt, bb)
    cx, cy, cz, kk, radius = group
    crg = cr // kk
    return pl.pallas_call(
        functools.partial(_group_lin_body, kk=kk, radius=radius),
        grid=grid,
        in_specs=[
            pl.BlockSpec((cr, cin_pad), lambda i: (i, 0)),
            pl.BlockSpec((crg, 1), lambda i: (i, 0)),
            pl.BlockSpec((crg, 1), lambda i: (i, 0)),
            pl.BlockSpec((crg, 1), lambda i: (i, 0)),
            pl.BlockSpec((cin_pad, cout), lambda i: (0, 0)),
            _vec_spec(cout),
        ],
        out_specs=out_specs,
        out_shape=out_shape,
    )(x, cx, cy, cz, wt, bb)


def _normlin_call(y, s, q, g, bt, w, bias):
    rows, cin = y.shape
    cout = w.shape[0]
    cr = _chunk_rows(rows, max(cin, cout))
    grid = (rows // cr,)
    wt = _prep_w(w, cin)
    bb = bias.reshape(1, cout)
    return pl.pallas_call(
        functools.partial(_normlin_body, nrows=rows),
        grid=grid,
        in_specs=[
            pl.BlockSpec((cr, cin), lambda i: (i, 0)),
            _vec_spec(cin),
            _vec_spec(cin),
            _vec_spec(cin),
            _vec_spec(cin),
            pl.BlockSpec((cin, cout), lambda i: (0, 0)),
            _vec_spec(cout),
        ],
        out_specs=[
            pl.BlockSpec((cr, cout), lambda i: (i, 0)),
            _vec_spec(cout),
            _vec_spec(cout),
        ],
        out_shape=[
            jax.ShapeDtypeStruct((rows, cout), _F32),
            jax.ShapeDtypeStruct((1, cout), _F32),
            jax.ShapeDtypeStruct((1, cout), _F32),
        ],
    )(y, s, q, g.reshape(1, cin), bt.reshape(1, cin), wt, bb)


def _pool_call(y, s, q, g, bt, kk):
    rows, cin = y.shape
    cr = _chunk_rows(rows, cin)
    grid = (rows // cr,)
    return pl.pallas_call(
        functools.partial(_pool_body, nrows=rows, kk=kk),
        grid=grid,
        in_specs=[
            pl.BlockSpec((cr, cin), lambda i: (i, 0)),
            _vec_spec(cin),
            _vec_spec(cin),
            _vec_spec(cin),
            _vec_spec(cin),
        ],
        out_specs=pl.BlockSpec((cr // kk, cin), lambda i: (i, 0)),
        out_shape=jax.ShapeDtypeStruct((rows // kk, cin), _F32),
    )(y, s, q, g.reshape(1, cin), bt.reshape(1, cin))


def _finalize_call(y, s, q, g, bt):
    rows, cin = y.shape
    cr = _chunk_rows(rows, cin)
    grid = (rows // cr,)
    return pl.pallas_call(
        functools.partial(_finalize_body, nrows=rows),
        grid=grid,
        in_specs=[
            pl.BlockSpec((cr, cin), lambda i: (i, 0)),
            _vec_spec(cin),
            _vec_spec(cin),
            _vec_spec(cin),
            _vec_spec(cin),
        ],
        out_specs=pl.BlockSpec((cr, cin), lambda i: (i, 0)),
        out_shape=jax.ShapeDtypeStruct((rows, cin), _F32),
    )(y, s, q, g.reshape(1, cin), bt.reshape(1, cin))


def _head_call(y, s, q, g, bt, w2, b2):
    rows, cin = y.shape
    cout = w2.shape[0]
    cr = _chunk_rows(rows, cin)
    grid = (rows // cr,)
    wt = _prep_w(w2, cin)
    bb = b2.reshape(1, cout)
    return pl.pallas_call(
        functools.partial(_head_body, nrows=rows),
        grid=grid,
        in_specs=[
            pl.BlockSpec((cr, cin), lambda i: (i, 0)),
            _vec_spec(cin),
            _vec_spec(cin),
            _vec_spec(cin),
            _vec_spec(cin),
            pl.BlockSpec((cin, cout), lambda i: (0, 0)),
            _vec_spec(cout),
        ],
        out_specs=pl.BlockSpec((cr, cout), lambda i: (i, 0)),
        out_shape=jax.ShapeDtypeStruct((rows, cout), _F32),
    )(y, s, q, g.reshape(1, cin), bt.reshape(1, cin), wt, bb)


def _fp_body(x1_ref, y1_ref, z1_ref, x2_ref, y2_ref, z2_ref, p2_ref, o_ref,
             *, n1, n2):
    x1 = x1_ref[...].reshape(n1, 1)
    y1 = y1_ref[...].reshape(n1, 1)
    z1 = z1_ref[...].reshape(n1, 1)
    x2 = x2_ref[...].reshape(1, n2)
    y2 = y2_ref[...].reshape(1, n2)
    z2 = z2_ref[...].reshape(1, n2)
    dx = x1 - x2
    dy = y1 - y2
    dz = z1 - z2
    d = dx * dx + dy * dy + dz * dz
    iota = lax.broadcasted_iota(jnp.int32, (n1, n2), 1)
    picks = []
    for _ in range(3):
        mv = jnp.min(d, axis=1, keepdims=True)
        am = jnp.min(jnp.where(d <= mv, iota, n2), axis=1, keepdims=True)
        picks.append((1.0 / (mv + 1e-8), am))
        d = jnp.where(iota == am, jnp.inf, d)
    norm = picks[0][0] + picks[1][0] + picks[2][0]
    wm = jnp.zeros((n1, n2), _F32)
    for rec, am in picks:
        wm = wm + (iota == am).astype(_F32) * (rec / norm)
    o_ref[...] = jnp.dot(wm, p2_ref[0], preferred_element_type=_F32)


def _fp_interp(x1, y1, z1, x2, y2, z2, p2):
    b, n1 = x1.shape
    n2 = x2.shape[1]
    c2 = p2.shape[2]
    return pl.pallas_call(
        functools.partial(_fp_body, n1=n1, n2=n2),
        grid=(b,),
        in_specs=[
            pl.BlockSpec((1, 1, n1), lambda i: (i, 0, 0)),
            pl.BlockSpec((1, 1, n1), lambda i: (i, 0, 0)),
            pl.BlockSpec((1, 1, n1), lambda i: (i, 0, 0)),
            pl.BlockSpec((1, 1, n2), lambda i: (i, 0, 0)),
            pl.BlockSpec((1, 1, n2), lambda i: (i, 0, 0)),
            pl.BlockSpec((1, 1, n2), lambda i: (i, 0, 0)),
            pl.BlockSpec((1, n2, c2), lambda i: (i, 0, 0)),
        ],
        out_specs=pl.BlockSpec((n1, c2), lambda i: (i, 0)),
        out_shape=jax.ShapeDtypeStruct((b * n1, c2), _F32),
    )(x1.reshape(b, 1, n1), y1.reshape(b, 1, n1), z1.reshape(b, 1, n1),
      x2.reshape(b, 1, n2), y2.reshape(b, 1, n2), z2.reshape(b, 1, n2), p2)


def _pad_cols(a, cpad):
    rows, c = a.shape
    if c == cpad:
        return a
    return jnp.concatenate([a, jnp.zeros((rows, cpad - c), _F32)], axis=1)


def _set_abstraction(x, y, z, feats, npoint, radius, nsample, layers, cs):
    b, n = x.shape
    cfeat = feats.shape[2]
    cx, cy, cz = _fps(x, y, z, npoint)
    gidx = _ball_query(x, y, z, cx, cy, cz, radius, nsample, cs)
    cin = 3 + cfeat
    cpad = ((cin + 15) // 16) * 16
    table = jnp.concatenate(
        [x[..., None], y[..., None], z[..., None], feats], axis=2)
    table = _pad_cols(table.reshape(b * n, cin), cpad)
    rows = _sc_gather(table, gidx.reshape(-1), cpad)
    cenx = cx.reshape(b * npoint, 1)
    ceny = cy.reshape(b * npoint, 1)
    cenz = cz.reshape(b * npoint, 1)
    yv, s, q = _lin_call(rows, layers[0][0], layers[0][1],
                         group=(cenx, ceny, cenz, nsample, radius))
    g, bt = layers[0][2], layers[0][3]
    for (w, bias, g2, bt2) in layers[1:]:
        yv, s2, q2 = _normlin_call(yv, s, q, g, bt, w, bias)
        s, q, g, bt = s2, q2, g2, bt2
    pooled = _pool_call(yv, s, q, g, bt, nsample)
    return cx, cy, cz, pooled


def _feature_prop(x1, y1, z1, x2, y2, z2, p1flat, p2, layers, finalize):
    b, n1 = x1.shape
    interp = _fp_interp(x1, y1, z1, x2, y2, z2, p2)
    xin = jnp.concatenate([p1flat, interp], axis=1)
    yv, s, q = _lin_call(xin, layers[0][0], layers[0][1])
    g, bt = layers[0][2], layers[0][3]
    for (w, bias, g2, bt2) in layers[1:]:
        yv, s2, q2 = _normlin_call(yv, s, q, g, bt, w, bias)
        s, q, g, bt = s2, q2, g2, bt2
    if finalize:
        return _finalize_call(yv, s, q, g, bt)
    return yv, s, q, g, bt


def kernel(xyz, features, params):
    b, n, _ = xyz.shape
    x0 = xyz[..., 0]
    y0 = xyz[..., 1]
    z0 = xyz[..., 2]

    cx1, cy1, cz1, p1 = _set_abstraction(
        x0, y0, z0, features, 256, 0.05, 32, params['sa1'], cs=32)
    p1r = p1.reshape(b, 256, p1.shape[1])
    cx2, cy2, cz2, p2 = _set_abstraction(
        cx1, cy1, cz1, p1r, 128, 0.1, 64, params['sa2'], cs=128)
    p2r = p2.reshape(b, 128, p2.shape[1])
    cx3, cy3, cz3, p3 = _set_abstraction(
        cx2, cy2, cz2, p2r, 64, 0.2, 128, params['sa3'], cs=64)
    p3r = p3.reshape(b, 64, p3.shape[1])

    l2f = _feature_prop(cx2, cy2, cz2, cx3, cy3, cz3, p2, p3r,
                        params['fp3'], finalize=True)
    l2fr = l2f.reshape(b, 128, l2f.shape[1])
    l1f = _feature_prop(cx1, cy1, cz1, cx2, cy2, cz2, p1, l2fr,
                        params['fp2'], finalize=True)
    l1fr = l1f.reshape(b, 256, l1f.shape[1])
    f0flat = features.reshape(b * n, features.shape[2])
    yv, s, q, g, bt = _feature_prop(x0, y0, z0, cx1, cy1, cz1, f0flat, l1fr,
                                    params['fp1'], finalize=False)

    w1, b1, g1, bt1 = params['head1']
    yh, sh, qh = _normlin_call(yv, s, q, g, bt, w1, b1)
    w2, b2 = params['head2']
    out = _head_call(yh, sh, qh, g1, bt1, w2, b2)
    return out.reshape(b, n)


# speedup vs baseline: 1.2499x; 1.0943x over previous
import functools

import jax
import jax.numpy as jnp
from jax import lax
from jax.experimental import pallas as pl
from jax.experimental.pallas import tpu as pltpu
from jax.experimental.pallas import tpu_sc as plsc

_F32 = jnp.float32


def _fps_body(x_ref, y_ref, z_ref, ox_ref, oy_ref, oz_ref, *, npoint):
    x = x_ref[...]
    y = y_ref[...]
    z = z_ref[...]
    b, n = x.shape
    iota = lax.broadcasted_iota(jnp.int32, (b, n), 1)
    siota = lax.broadcasted_iota(jnp.int32, (b, npoint), 1)

    def body(s, carry):
        dist, cx, cy, cz, ax, ay, az = carry
        sel = siota == s
        ax = jnp.where(sel, cx, ax)
        ay = jnp.where(sel, cy, ay)
        az = jnp.where(sel, cz, az)
        d = (x - cx) ** 2 + (y - cy) ** 2 + (z - cz) ** 2
        dist = jnp.minimum(dist, d)
        mx = jnp.max(dist, axis=1, keepdims=True)
        am = jnp.min(jnp.where(dist >= mx, iota, n), axis=1, keepdims=True)
        oh = (iota == am).astype(_F32)
        ncx = jnp.sum(x * oh, axis=1, keepdims=True)
        ncy = jnp.sum(y * oh, axis=1, keepdims=True)
        ncz = jnp.sum(z * oh, axis=1, keepdims=True)
        return dist, ncx, ncy, ncz, ax, ay, az

    init = (
        jnp.full((b, n), 1e10, _F32),
        x[:, 0:1],
        y[:, 0:1],
        z[:, 0:1],
        jnp.zeros((b, npoint), _F32),
        jnp.zeros((b, npoint), _F32),
        jnp.zeros((b, npoint), _F32),
    )
    _, _, _, _, ax, ay, az = lax.fori_loop(0, npoint, body, init)
    ox_ref[...] = ax
    oy_ref[...] = ay
    oz_ref[...] = az


def _fps(x, y, z, npoint):
    b, n = x.shape
    return pl.pallas_call(
        functools.partial(_fps_body, npoint=npoint),
        out_shape=[jax.ShapeDtypeStruct((b, npoint), _F32)] * 3,
    )(x, y, z)


def _bq_body(xr_ref, yr_ref, zr_ref, cx_ref, cy_ref, cz_ref, o_ref, *, n, nb,
             cs, k, r2):
    xr = xr_ref[0][None]
    yr = yr_ref[0][None]
    zr = zr_ref[0][None]
    cx = cx_ref[...].reshape(cs, 1, 1)
    cy = cy_ref[...].reshape(cs, 1, 1)
    cz = cz_ref[...].reshape(cs, 1, 1)
    dx = cx - xr
    dy = cy - yr
    dz = cz - zr
    d = dx * dx + dy * dy + dz * dz
    maskf = (d <= r2).astype(_F32)
    u = (lax.broadcasted_iota(jnp.int32, (128, 128), 0)
         <= lax.broadcasted_iota(jnp.int32, (128, 128), 1)).astype(_F32)
    within = jnp.dot(maskf.reshape(cs * nb, 128), u,
                     preferred_element_type=_F32).reshape(cs, nb, 128)
    bsum = within[:, :, 127:128].reshape(cs, nb)
    us = (lax.broadcasted_iota(jnp.int32, (nb, nb), 0)
          < lax.broadcasted_iota(jnp.int32, (nb, nb), 1)).astype(_F32)
    offs = jnp.dot(bsum, us, preferred_element_type=_F32)
    m = jnp.sum(bsum, axis=1, keepdims=True)
    cnt = (within + offs[:, :, None]).reshape(cs, n)
    kk = lax.broadcasted_iota(jnp.int32, (cs, k), 1).astype(_F32)
    keff = jnp.where(kk + 1.0 <= m, kk, 0.0)
    cmp = (cnt[:, None, :] <= keff[:, :, None]).astype(_F32)
    idxf = jnp.sum(cmp, axis=2)
    b_id = pl.program_id(0)
    o_ref[...] = idxf.astype(jnp.int32) + b_id * n


def _ball_query(x, y, z, cx, cy, cz, radius, k, cs):
    b, n = x.shape
    s = cx.shape[1]
    nb = n // 128
    xr = x.reshape(b, nb, 128)
    yr = y.reshape(b, nb, 128)
    zr = z.reshape(b, nb, 128)
    nch = s // cs
    grid = (b, nch)
    cx3 = cx.reshape(b * nch, 1, cs)
    cy3 = cy.reshape(b * nch, 1, cs)
    cz3 = cz.reshape(b * nch, 1, cs)
    return pl.pallas_call(
        functools.partial(_bq_body, n=n, nb=nb, cs=cs, k=k, r2=radius * radius),
        grid=grid,
        in_specs=[
            pl.BlockSpec((1, nb, 128), lambda i, j: (i, 0, 0)),
            pl.BlockSpec((1, nb, 128), lambda i, j: (i, 0, 0)),
            pl.BlockSpec((1, nb, 128), lambda i, j: (i, 0, 0)),
            pl.BlockSpec((1, 1, cs), lambda i, j: (i * nch + j, 0, 0)),
            pl.BlockSpec((1, 1, cs), lambda i, j: (i * nch + j, 0, 0)),
            pl.BlockSpec((1, 1, cs), lambda i, j: (i * nch + j, 0, 0)),
        ],
        out_specs=pl.BlockSpec((cs, k), lambda i, j: (i * nch + j, 0)),
        out_shape=jax.ShapeDtypeStruct((b * s, k), jnp.int32),
    )(xr, yr, zr, cx3, cy3, cz3)


def _sc_gather(table, gidx, d_pad, tc_tiling):
    r = gidx.shape[0]
    nrows_t = table.shape[0]
    nw = 32
    rpw = r // nw
    ch = 8
    while ch * 2 <= rpw and ch * 2 * d_pad * 4 <= 300 * 1024:
        ch *= 2
    nchunks = rpw // ch
    mesh = plsc.VectorSubcoreMesh(core_axis_name="c", subcore_axis_name="s")

    @functools.partial(
        pl.kernel,
        mesh=mesh,
        out_type=jax.ShapeDtypeStruct((r, d_pad), _F32),
        scratch_types=[
            pltpu.VMEM((ch,), jnp.int32),
            pltpu.VMEM((ch, d_pad), _F32),
            pltpu.VMEM_SHARED((nrows_t, d_pad), _F32),
            pltpu.SemaphoreType.DMA,
        ],
        compiler_params=pltpu.CompilerParams(use_tc_tiling_on_sc=tc_tiling),
    )
    def gather_k(table_hbm, idx_hbm, out_hbm, idx_v, rows_v, table_sh, sem):
        sid = lax.axis_index("s")
        wid = sid * 2 + lax.axis_index("c")
        base = wid * rpw

        @pl.when(sid == 0)
        def _():
            pltpu.sync_copy(table_hbm, table_sh)

        plsc.subcore_barrier()

        def body(c, carry):
            off = base + c * ch
            pltpu.sync_copy(idx_hbm.at[pl.ds(off, ch)], idx_v)
            pltpu.async_copy(table_sh.at[idx_v], rows_v, sem).wait()
            pltpu.sync_copy(rows_v, out_hbm.at[pl.ds(off, ch)])
            return carry

        lax.fori_loop(0, nchunks, body, 0)

    return gather_k(table, gidx)


def _accum_stats(y, s_ref, q_ref):
    @pl.when(pl.program_id(0) == 0)
    def _():
        s_ref[...] = jnp.zeros_like(s_ref)
        q_ref[...] = jnp.zeros_like(q_ref)

    s_ref[...] += jnp.sum(y, axis=0, keepdims=True)
    q_ref[...] += jnp.sum(y * y, axis=0, keepdims=True)


def _norm_consts(s_ref, q_ref, g_ref, bt_ref, nrows):
    mu = s_ref[...] * (1.0 / nrows)
    var = jnp.maximum(q_ref[...] * (1.0 / nrows) - mu * mu, 0.0)
    sc = lax.rsqrt(var + 1e-5) * g_ref[...]
    return mu, sc, bt_ref[...]


def _group_lin_body(g_ref, cx_ref, cy_ref, cz_ref, wt_ref, bb_ref, y_ref,
                    s_ref, q_ref, *, kk, radius):
    g = g_ref[...]
    crg = g.shape[0] // kk
    dpad = g.shape[1]
    g3 = g.reshape(crg, kk, dpad)
    lane = lax.broadcasted_iota(jnp.int32, (crg, 1, dpad), 2)
    cx = cx_ref[...].reshape(crg, 1, 1)
    cy = cy_ref[...].reshape(crg, 1, 1)
    cz = cz_ref[...].reshape(crg, 1, 1)
    off = jnp.where(lane == 0, -cx,
                    jnp.where(lane == 1, -cy,
                              jnp.where(lane == 2, -cz, 0.0)))
    dv = jnp.where(lane < 3, radius, 1.0)
    xin = ((g3 + off) / dv).reshape(crg * kk, dpad)
    y = jnp.dot(xin, wt_ref[...], preferred_element_type=_F32) + bb_ref[...]
    y_ref[...] = y
    _accum_stats(y, s_ref, q_ref)


def _lin_body(x_ref, wt_ref, bb_ref, y_ref, s_ref, q_ref):
    y = jnp.dot(x_ref[...], wt_ref[...],
                preferred_element_type=_F32) + bb_ref[...]
    y_ref[...] = y
    _accum_stats(y, s_ref, q_ref)


def _normlin_body(y_ref, si_ref, qi_ref, g_ref, bt_ref, wt_ref, bb_ref,
                  o_ref, s_ref, q_ref, *, nrows):
    mu, sc, bt = _norm_consts(si_ref, qi_ref, g_ref, bt_ref, nrows)
    x = jnp.maximum((y_ref[...] - mu) * sc + bt, 0.0)
    y = jnp.dot(x, wt_ref[...], preferred_element_type=_F32) + bb_ref[...]
    o_ref[...] = y
    _accum_stats(y, s_ref, q_ref)


def _normlin_max_body(y_ref, si_ref, qi_ref, g_ref, bt_ref, wt_ref, bb_ref,
                      m_ref, s_ref, q_ref, *, nrows, kk):
    mu, sc, bt = _norm_consts(si_ref, qi_ref, g_ref, bt_ref, nrows)
    x = jnp.maximum((y_ref[...] - mu) * sc + bt, 0.0)
    y = jnp.dot(x, wt_ref[...], preferred_element_type=_F32) + bb_ref[...]
    rg = y.shape[0] // kk
    m_ref[...] = jnp.max(y.reshape(rg, kk, y.shape[1]), axis=1)
    _accum_stats(y, s_ref, q_ref)


def _finalize_body(y_ref, si_ref, qi_ref, g_ref, bt_ref, o_ref, *, nrows):
    mu, sc, bt = _norm_consts(si_ref, qi_ref, g_ref, bt_ref, nrows)
    o_ref[...] = jnp.maximum((y_ref[...] - mu) * sc + bt, 0.0)


def _head_body(y_ref, si_ref, qi_ref, g_ref, bt_ref, wt_ref, bb_ref, o_ref,
               *, nrows):
    mu, sc, bt = _norm_consts(si_ref, qi_ref, g_ref, bt_ref, nrows)
    x = jnp.maximum((y_ref[...] - mu) * sc + bt, 0.0)
    z = jnp.dot(x, wt_ref[...], preferred_element_type=_F32) + bb_ref[...]
    o_ref[...] = jax.nn.sigmoid(z)


def _prep_w(w, cin_pad):
    cout, cin = w.shape
    wt = jnp.zeros((cin_pad, cout), _F32)
    return wt.at[:cin, :].set(w.T)


def _chunk_rows(rows, cmax):
    c = 8192 if cmax <= 256 else 4096
    return min(rows, c)


def _vec_spec(cout):
    return pl.BlockSpec((1, cout), lambda i: (0, 0))


def _lin_call(x, w, bias, *, group=None):
    rows, cin_pad = x.shape
    cout = w.shape[0]
    cr = _chunk_rows(rows, max(cin_pad, cout))
    grid = (rows // cr,)
    wt = _prep_w(w, cin_pad)
    bb = bias.reshape(1, cout)
    out_shape = [
        jax.ShapeDtypeStruct((rows, cout), _F32),
        jax.ShapeDtypeStruct((1, cout), _F32),
        jax.ShapeDtypeStruct((1, cout), _F32),
    ]
    out_specs = [
        pl.BlockSpec((cr, cout), lambda i: (i, 0)),
        _vec_spec(cout),
        _vec_spec(cout),
    ]
    if group is None:
        return pl.pallas_call(
            _lin_body,
            grid=grid,
            in_specs=[
                pl.BlockSpec((cr, cin_pad), lambda i: (i, 0)),
                pl.BlockSpec((cin_pad, cout), lambda i: (0, 0)),
                _vec_spec(cout),
            ],
            out_specs=out_specs,
            out_shape=out_shape,
        )(x, wt, bb)
    cx, cy, cz, kk, radius = group
    crg = cr // kk
    return pl.pallas_call(
        functools.partial(_group_lin_body, kk=kk, radius=radius),
        grid=grid,
        in_specs=[
            pl.BlockSpec((cr, cin_pad), lambda i: (i, 0)),
            pl.BlockSpec((crg, 1), lambda i: (i, 0)),
            pl.BlockSpec((crg, 1), lambda i: (i, 0)),
            pl.BlockSpec((crg, 1), lambda i: (i, 0)),
            pl.BlockSpec((cin_pad, cout), lambda i: (0, 0)),
            _vec_spec(cout),
        ],
        out_specs=out_specs,
        out_shape=out_shape,
    )(x, cx, cy, cz, wt, bb)


def _normlin_call(y, s, q, g, bt, w, bias):
    rows, cin = y.shape
    cout = w.shape[0]
    cr = _chunk_rows(rows, max(cin, cout))
    grid = (rows // cr,)
    wt = _prep_w(w, cin)
    bb = bias.reshape(1, cout)
    return pl.pallas_call(
        functools.partial(_normlin_body, nrows=rows),
        grid=grid,
        in_specs=[
            pl.BlockSpec((cr, cin), lambda i: (i, 0)),
            _vec_spec(cin),
            _vec_spec(cin),
            _vec_spec(cin),
            _vec_spec(cin),
            pl.BlockSpec((cin, cout), lambda i: (0, 0)),
            _vec_spec(cout),
        ],
        out_specs=[
            pl.BlockSpec((cr, cout), lambda i: (i, 0)),
            _vec_spec(cout),
            _vec_spec(cout),
        ],
        out_shape=[
            jax.ShapeDtypeStruct((rows, cout), _F32),
            jax.ShapeDtypeStruct((1, cout), _F32),
            jax.ShapeDtypeStruct((1, cout), _F32),
        ],
    )(y, s, q, g.reshape(1, cin), bt.reshape(1, cin), wt, bb)


def _normlin_max_call(y, s, q, g, bt, w, bias, kk):
    rows, cin = y.shape
    cout = w.shape[0]
    cr = _chunk_rows(rows, max(cin, cout))
    grid = (rows // cr,)
    wt = _prep_w(w, cin)
    bb = bias.reshape(1, cout)
    return pl.pallas_call(
        functools.partial(_normlin_max_body, nrows=rows, kk=kk),
        grid=grid,
        in_specs=[
            pl.BlockSpec((cr, cin), lambda i: (i, 0)),
            _vec_spec(cin),
            _vec_spec(cin),
            _vec_spec(cin),
            _vec_spec(cin),
            pl.BlockSpec((cin, cout), lambda i: (0, 0)),
            _vec_spec(cout),
        ],
        out_specs=[
            pl.BlockSpec((cr // kk, cout), lambda i: (i, 0)),
            _vec_spec(cout),
            _vec_spec(cout),
        ],
        out_shape=[
            jax.ShapeDtypeStruct((rows // kk, cout), _F32),
            jax.ShapeDtypeStruct((1, cout), _F32),
            jax.ShapeDtypeStruct((1, cout), _F32),
        ],
    )(y, s, q, g.reshape(1, cin), bt.reshape(1, cin), wt, bb)


def _finalize_call(y, s, q, g, bt, nrows=None):
    rows, cin = y.shape
    cr = _chunk_rows(rows, cin)
    grid = (rows // cr,)
    return pl.pallas_call(
        functools.partial(_finalize_body, nrows=nrows if nrows else rows),
        grid=grid,
        in_specs=[
            pl.BlockSpec((cr, cin), lambda i: (i, 0)),
            _vec_spec(cin),
            _vec_spec(cin),
            _vec_spec(cin),
            _vec_spec(cin),
        ],
        out_specs=pl.BlockSpec((cr, cin), lambda i: (i, 0)),
        out_shape=jax.ShapeDtypeStruct((rows, cin), _F32),
    )(y, s, q, g.reshape(1, cin), bt.reshape(1, cin))


def _head_call(y, s, q, g, bt, w2, b2):
    rows, cin = y.shape
    cout = w2.shape[0]
    cr = _chunk_rows(rows, cin)
    grid = (rows // cr,)
    wt = _prep_w(w2, cin)
    bb = b2.reshape(1, cout)
    return pl.pallas_call(
        functools.partial(_head_body, nrows=rows),
        grid=grid,
        in_specs=[
            pl.BlockSpec((cr, cin), lambda i: (i, 0)),
            _vec_spec(cin),
            _vec_spec(cin),
            _vec_spec(cin),
            _vec_spec(cin),
            pl.BlockSpec((cin, cout), lambda i: (0, 0)),
            _vec_spec(cout),
        ],
        out_specs=pl.BlockSpec((cr, cout), lambda i: (i, 0)),
        out_shape=jax.ShapeDtypeStruct((rows, cout), _F32),
    )(y, s, q, g.reshape(1, cin), bt.reshape(1, cin), wt, bb)


def _fp_body(x1_ref, y1_ref, z1_ref, x2_ref, y2_ref, z2_ref, p2_ref, o_ref,
             *, n1, n2):
    x1 = x1_ref[...].reshape(n1, 1)
    y1 = y1_ref[...].reshape(n1, 1)
    z1 = z1_ref[...].reshape(n1, 1)
    x2 = x2_ref[...].reshape(1, n2)
    y2 = y2_ref[...].reshape(1, n2)
    z2 = z2_ref[...].reshape(1, n2)
    dx = x1 - x2
    dy = y1 - y2
    dz = z1 - z2
    d = dx * dx + dy * dy + dz * dz
    iota = lax.broadcasted_iota(jnp.int32, (n1, n2), 1)
    picks = []
    for _ in range(3):
        mv = jnp.min(d, axis=1, keepdims=True)
        am = jnp.min(jnp.where(d <= mv, iota, n2), axis=1, keepdims=True)
        picks.append((1.0 / (mv + 1e-8), am))
        d = jnp.where(iota == am, jnp.inf, d)
    norm = picks[0][0] + picks[1][0] + picks[2][0]
    wm = jnp.zeros((n1, n2), _F32)
    for rec, am in picks:
        wm = wm + (iota == am).astype(_F32) * (rec / norm)
    o_ref[...] = jnp.dot(wm, p2_ref[0], preferred_element_type=_F32)


def _fp_interp(x1, y1, z1, x2, y2, z2, p2):
    b, n1 = x1.shape
    n2 = x2.shape[1]
    c2 = p2.shape[2]
    return pl.pallas_call(
        functools.partial(_fp_body, n1=n1, n2=n2),
        grid=(b,),
        in_specs=[
            pl.BlockSpec((1, 1, n1), lambda i: (i, 0, 0)),
            pl.BlockSpec((1, 1, n1), lambda i: (i, 0, 0)),
            pl.BlockSpec((1, 1, n1), lambda i: (i, 0, 0)),
            pl.BlockSpec((1, 1, n2), lambda i: (i, 0, 0)),
            pl.BlockSpec((1, 1, n2), lambda i: (i, 0, 0)),
            pl.BlockSpec((1, 1, n2), lambda i: (i, 0, 0)),
            pl.BlockSpec((1, n2, c2), lambda i: (i, 0, 0)),
        ],
        out_specs=pl.BlockSpec((n1, c2), lambda i: (i, 0)),
        out_shape=jax.ShapeDtypeStruct((b * n1, c2), _F32),
    )(x1.reshape(b, 1, n1), y1.reshape(b, 1, n1), z1.reshape(b, 1, n1),
      x2.reshape(b, 1, n2), y2.reshape(b, 1, n2), z2.reshape(b, 1, n2), p2)


def _pad_cols(a, cpad):
    rows, c = a.shape
    if c == cpad:
        return a
    return jnp.concatenate([a, jnp.zeros((rows, cpad - c), _F32)], axis=1)


def _set_abstraction(x, y, z, feats, npoint, radius, nsample, layers, cs):
    b, n = x.shape
    cfeat = feats.shape[2]
    cx, cy, cz = _fps(x, y, z, npoint)
    gidx = _ball_query(x, y, z, cx, cy, cz, radius, nsample, cs)
    cin = 3 + cfeat
    cpad = ((cin + 15) // 16) * 16
    table = jnp.concatenate(
        [x[..., None], y[..., None], z[..., None], feats], axis=2)
    table = _pad_cols(table.reshape(b * n, cin), cpad)
    rows = _sc_gather(table, gidx.reshape(-1), cpad, False)
    cenx = cx.reshape(b * npoint, 1)
    ceny = cy.reshape(b * npoint, 1)
    cenz = cz.reshape(b * npoint, 1)
    yv, s, q = _lin_call(rows, layers[0][0], layers[0][1],
                         group=(cenx, ceny, cenz, nsample, radius))
    g, bt = layers[0][2], layers[0][3]
    nrows_full = yv.shape[0]
    for (w, bias, g2, bt2) in layers[1:-1]:
        yv, s2, q2 = _normlin_call(yv, s, q, g, bt, w, bias)
        s, q, g, bt = s2, q2, g2, bt2
    w, bias, g2, bt2 = layers[-1]
    m, s2, q2 = _normlin_max_call(yv, s, q, g, bt, w, bias, nsample)
    pooled = _finalize_call(m, s2, q2, g2, bt2, nrows=nrows_full)
    return cx, cy, cz, pooled


def _feature_prop(x1, y1, z1, x2, y2, z2, p1flat, p2, layers, finalize):
    b, n1 = x1.shape
    interp = _fp_interp(x1, y1, z1, x2, y2, z2, p2)
    xin = jnp.concatenate([p1flat, interp], axis=1)
    yv, s, q = _lin_call(xin, layers[0][0], layers[0][1])
    g, bt = layers[0][2], layers[0][3]
    for (w, bias, g2, bt2) in layers[1:]:
        yv, s2, q2 = _normlin_call(yv, s, q, g, bt, w, bias)
        s, q, g, bt = s2, q2, g2, bt2
    if finalize:
        return _finalize_call(yv, s, q, g, bt)
    return yv, s, q, g, bt


def kernel(xyz, features, params):
    b, n, _ = xyz.shape
    x0 = xyz[..., 0]
    y0 = xyz[..., 1]
    z0 = xyz[..., 2]

    cx1, cy1, cz1, p1 = _set_abstraction(
        x0, y0, z0, features, 256, 0.05, 32, params['sa1'], cs=32)
    p1r = p1.reshape(b, 256, p1.shape[1])
    cx2, cy2, cz2, p2 = _set_abstraction(
        cx1, cy1, cz1, p1r, 128, 0.1, 64, params['sa2'], cs=128)
    p2r = p2.reshape(b, 128, p2.shape[1])
    cx3, cy3, cz3, p3 = _set_abstraction(
        cx2, cy2, cz2, p2r, 64, 0.2, 128, params['sa3'], cs=64)
    p3r = p3.reshape(b, 64, p3.shape[1])

    l2f = _feature_prop(cx2, cy2, cz2, cx3, cy3, cz3, p2, p3r,
                        params['fp3'], finalize=True)
    l2fr = l2f.reshape(b, 128, l2f.shape[1])
    l1f = _feature_prop(cx1, cy1, cz1, cx2, cy2, cz2, p1, l2fr,
                        params['fp2'], finalize=True)
    l1fr = l1f.reshape(b, 256, l1f.shape[1])
    f0flat = features.reshape(b * n, features.shape[2])
    yv, s, q, g, bt = _feature_prop(x0, y0, z0, cx1, cy1, cz1, f0flat, l1fr,
                                    params['fp1'], finalize=False)

    w1, b1, g1, bt1 = params['head1']
    yh, sh, qh = _normlin_call(yv, s, q, g, bt, w1, b1)
    w2, b2 = params['head2']
    out = _head_call(yh, sh, qh, g1, bt1, w2, b2)
    return out.reshape(b, n)


# speedup vs baseline: 1.5445x; 1.2357x over previous
import functools

import jax
import jax.numpy as jnp
from jax import lax
from jax.experimental import pallas as pl
from jax.experimental.pallas import tpu as pltpu
from jax.experimental.pallas import tpu_sc as plsc

_F32 = jnp.float32


def _fps_body(x_ref, y_ref, z_ref, ox_ref, oy_ref, oz_ref, *, npoint):
    x = x_ref[...]
    y = y_ref[...]
    z = z_ref[...]
    b, n = x.shape
    iota = lax.broadcasted_iota(jnp.int32, (b, n), 1)
    siota = lax.broadcasted_iota(jnp.int32, (b, npoint), 1)

    def body(s, carry):
        dist, cx, cy, cz, ax, ay, az = carry
        sel = siota == s
        ax = jnp.where(sel, cx, ax)
        ay = jnp.where(sel, cy, ay)
        az = jnp.where(sel, cz, az)
        d = (x - cx) ** 2 + (y - cy) ** 2 + (z - cz) ** 2
        dist = jnp.minimum(dist, d)
        mx = jnp.max(dist, axis=1, keepdims=True)
        am = jnp.min(jnp.where(dist >= mx, iota, n), axis=1, keepdims=True)
        oh = (iota == am).astype(_F32)
        ncx = jnp.sum(x * oh, axis=1, keepdims=True)
        ncy = jnp.sum(y * oh, axis=1, keepdims=True)
        ncz = jnp.sum(z * oh, axis=1, keepdims=True)
        return dist, ncx, ncy, ncz, ax, ay, az

    init = (
        jnp.full((b, n), 1e10, _F32),
        x[:, 0:1],
        y[:, 0:1],
        z[:, 0:1],
        jnp.zeros((b, npoint), _F32),
        jnp.zeros((b, npoint), _F32),
        jnp.zeros((b, npoint), _F32),
    )
    _, _, _, _, ax, ay, az = lax.fori_loop(0, npoint, body, init)
    ox_ref[...] = ax
    oy_ref[...] = ay
    oz_ref[...] = az


def _fps(x, y, z, npoint):
    b, n = x.shape
    return pl.pallas_call(
        functools.partial(_fps_body, npoint=npoint),
        out_shape=[jax.ShapeDtypeStruct((b, npoint), _F32)] * 3,
    )(x, y, z)


def _bq_body(xr_ref, yr_ref, zr_ref, cx_ref, cy_ref, cz_ref, o_ref, *, n, nb,
             cs, k, r2):
    xr = xr_ref[0][None]
    yr = yr_ref[0][None]
    zr = zr_ref[0][None]
    cx = cx_ref[...].reshape(cs, 1, 1)
    cy = cy_ref[...].reshape(cs, 1, 1)
    cz = cz_ref[...].reshape(cs, 1, 1)
    dx = cx - xr
    dy = cy - yr
    dz = cz - zr
    d = dx * dx + dy * dy + dz * dz
    maskf = (d <= r2).astype(_F32)
    u = (lax.broadcasted_iota(jnp.int32, (128, 128), 0)
         <= lax.broadcasted_iota(jnp.int32, (128, 128), 1)).astype(_F32)
    within = jnp.dot(maskf.reshape(cs * nb, 128), u,
                     preferred_element_type=_F32).reshape(cs, nb, 128)
    bsum = within[:, :, 127:128].reshape(cs, nb)
    us = (lax.broadcasted_iota(jnp.int32, (nb, nb), 0)
          < lax.broadcasted_iota(jnp.int32, (nb, nb), 1)).astype(_F32)
    offs = jnp.dot(bsum, us, preferred_element_type=_F32)
    m = jnp.sum(bsum, axis=1, keepdims=True)
    cnt = (within + offs[:, :, None]).reshape(cs, n)
    kk = lax.broadcasted_iota(jnp.int32, (cs, k), 1).astype(_F32)
    keff = jnp.where(kk + 1.0 <= m, kk, 0.0)
    cmp = (cnt[:, None, :] <= keff[:, :, None]).astype(_F32)
    idxf = jnp.sum(cmp, axis=2)
    b_id = pl.program_id(0)
    o_ref[...] = idxf.astype(jnp.int32) + b_id * n


def _ball_query(x, y, z, cx, cy, cz, radius, k, cs):
    b, n = x.shape
    s = cx.shape[1]
    nb = n // 128
    xr = x.reshape(b, nb, 128)
    yr = y.reshape(b, nb, 128)
    zr = z.reshape(b, nb, 128)
    nch = s // cs
    grid = (b, nch)
    cx3 = cx.reshape(b * nch, 1, cs)
    cy3 = cy.reshape(b * nch, 1, cs)
    cz3 = cz.reshape(b * nch, 1, cs)
    return pl.pallas_call(
        functools.partial(_bq_body, n=n, nb=nb, cs=cs, k=k, r2=radius * radius),
        grid=grid,
        in_specs=[
            pl.BlockSpec((1, nb, 128), lambda i, j: (i, 0, 0)),
            pl.BlockSpec((1, nb, 128), lambda i, j: (i, 0, 0)),
            pl.BlockSpec((1, nb, 128), lambda i, j: (i, 0, 0)),
            pl.BlockSpec((1, 1, cs), lambda i, j: (i * nch + j, 0, 0)),
            pl.BlockSpec((1, 1, cs), lambda i, j: (i * nch + j, 0, 0)),
            pl.BlockSpec((1, 1, cs), lambda i, j: (i * nch + j, 0, 0)),
        ],
        out_specs=pl.BlockSpec((cs, k), lambda i, j: (i * nch + j, 0)),
        out_shape=jax.ShapeDtypeStruct((b * s, k), jnp.int32),
    )(xr, yr, zr, cx3, cy3, cz3)


def _sc_gather(table, gidx, d_pad, tc_tiling):
    r = gidx.shape[0]
    nrows_t = table.shape[0]
    nw = 32
    rpw = r // nw
    ch = 8
    while ch * 2 <= rpw and ch * 2 * d_pad * 4 <= 300 * 1024:
        ch *= 2
    nchunks = rpw // ch
    mesh = plsc.VectorSubcoreMesh(core_axis_name="c", subcore_axis_name="s")

    @functools.partial(
        pl.kernel,
        mesh=mesh,
        out_type=jax.ShapeDtypeStruct((r, d_pad), _F32),
        scratch_types=[
            pltpu.VMEM((ch,), jnp.int32),
            pltpu.VMEM((ch, d_pad), _F32),
            pltpu.VMEM_SHARED((nrows_t, d_pad), _F32),
            pltpu.SemaphoreType.DMA,
        ],
        compiler_params=pltpu.CompilerParams(use_tc_tiling_on_sc=tc_tiling),
    )
    def gather_k(table_hbm, idx_hbm, out_hbm, idx_v, rows_v, table_sh, sem):
        sid = lax.axis_index("s")
        wid = sid * 2 + lax.axis_index("c")
        base = wid * rpw

        @pl.when(sid == 0)
        def _():
            pltpu.sync_copy(table_hbm, table_sh)

        plsc.subcore_barrier()

        def body(c, carry):
            off = base + c * ch
            pltpu.sync_copy(idx_hbm.at[pl.ds(off, ch)], idx_v)
            pltpu.async_copy(table_sh.at[idx_v], rows_v, sem).wait()
            pltpu.sync_copy(rows_v, out_hbm.at[pl.ds(off, ch)])
            return carry

        lax.fori_loop(0, nchunks, body, 0)

    return gather_k(table, gidx)


def _accum_stats(y, s_ref, q_ref):
    @pl.when(pl.program_id(0) == 0)
    def _():
        s_ref[...] = jnp.zeros_like(s_ref)
        q_ref[...] = jnp.zeros_like(q_ref)

    s_ref[...] += jnp.sum(y, axis=0, keepdims=True)
    q_ref[...] += jnp.sum(y * y, axis=0, keepdims=True)


def _norm_consts(s_ref, q_ref, g_ref, bt_ref, nrows):
    mu = s_ref[...] * (1.0 / nrows)
    var = jnp.maximum(q_ref[...] * (1.0 / nrows) - mu * mu, 0.0)
    sc = lax.rsqrt(var + 1e-5) * g_ref[...]
    return mu, sc, bt_ref[...]


def _group_lin_body(g_ref, cx_ref, cy_ref, cz_ref, wt_ref, bb_ref, y_ref,
                    s_ref, q_ref, *, kk, radius):
    g = g_ref[...]
    crg = g.shape[0] // kk
    dpad = g.shape[1]
    g3 = g.reshape(crg, kk, dpad)
    lane = lax.broadcasted_iota(jnp.int32, (crg, 1, dpad), 2)
    cx = cx_ref[...].reshape(crg, 1, 1)
    cy = cy_ref[...].reshape(crg, 1, 1)
    cz = cz_ref[...].reshape(crg, 1, 1)
    off = jnp.where(lane == 0, -cx,
                    jnp.where(lane == 1, -cy,
                              jnp.where(lane == 2, -cz, 0.0)))
    dv = jnp.where(lane < 3, radius, 1.0)
    xin = ((g3 + off) / dv).reshape(crg * kk, dpad)
    y = jnp.dot(xin, wt_ref[...], preferred_element_type=_F32) + bb_ref[...]
    y_ref[...] = y
    _accum_stats(y, s_ref, q_ref)


def _lin_body(x_ref, wt_ref, bb_ref, y_ref, s_ref, q_ref):
    y = jnp.dot(x_ref[...], wt_ref[...],
                preferred_element_type=_F32) + bb_ref[...]
    y_ref[...] = y
    _accum_stats(y, s_ref, q_ref)


def _nostat_lin_body(x_ref, wt_ref, bb_ref, y_ref):
    y_ref[...] = jnp.dot(x_ref[...], wt_ref[...],
                         preferred_element_type=_F32) + bb_ref[...]


def _nostat_lin(x, w, bias):
    rows, cin = x.shape
    cout = w.shape[0]
    cr = _chunk_rows(rows, max(cin, cout))
    grid = (rows // cr,)
    wt = _prep_w(w, cin)
    bb = bias.reshape(1, cout)
    return pl.pallas_call(
        _nostat_lin_body,
        grid=grid,
        in_specs=[
            pl.BlockSpec((cr, cin), lambda i: (i, 0)),
            pl.BlockSpec((cin, cout), lambda i: (0, 0)),
            _vec_spec(cout),
        ],
        out_specs=pl.BlockSpec((cr, cout), lambda i: (i, 0)),
        out_shape=jax.ShapeDtypeStruct((rows, cout), _F32),
    )(x, wt, bb)


def _onehot_group_body(idx_ref, a_ref, d_ref, y_ref, s_ref, q_ref, *, n, kk):
    b_id = pl.program_id(0)
    idx = idx_ref[...]
    srows = idx.shape[0]
    il = (idx - b_id * n)[:, :, None]
    iot = lax.broadcasted_iota(jnp.int32, (srows, kk, n), 2)
    oh = (iot == il).astype(_F32).reshape(srows * kk, n)
    a = a_ref[0]
    cout = a.shape[1]
    y = jnp.dot(oh, a, preferred_element_type=_F32)
    y = (y.reshape(srows, kk, cout)
         + d_ref[...].reshape(srows, 1, cout)).reshape(srows * kk, cout)
    y_ref[...] = y
    _accum_stats(y, s_ref, q_ref)


def _onehot_group(gidx, arows, drows, kk):
    bs, k = gidx.shape
    b, n, cout = arows.shape
    s = bs // b
    return pl.pallas_call(
        functools.partial(_onehot_group_body, n=n, kk=kk),
        grid=(b,),
        in_specs=[
            pl.BlockSpec((s, k), lambda i: (i, 0)),
            pl.BlockSpec((1, n, cout), lambda i: (i, 0, 0)),
            pl.BlockSpec((s, cout), lambda i: (i, 0)),
        ],
        out_specs=[
            pl.BlockSpec((s * k, cout), lambda i: (i, 0)),
            _vec_spec(cout),
            _vec_spec(cout),
        ],
        out_shape=[
            jax.ShapeDtypeStruct((bs * k, cout), _F32),
            jax.ShapeDtypeStruct((1, cout), _F32),
            jax.ShapeDtypeStruct((1, cout), _F32),
        ],
    )(gidx, arows, drows)


def _normlin_body(y_ref, si_ref, qi_ref, g_ref, bt_ref, wt_ref, bb_ref,
                  o_ref, s_ref, q_ref, *, nrows):
    mu, sc, bt = _norm_consts(si_ref, qi_ref, g_ref, bt_ref, nrows)
    x = jnp.maximum((y_ref[...] - mu) * sc + bt, 0.0)
    y = jnp.dot(x, wt_ref[...], preferred_element_type=_F32) + bb_ref[...]
    o_ref[...] = y
    _accum_stats(y, s_ref, q_ref)


def _normlin_max_body(y_ref, si_ref, qi_ref, g_ref, bt_ref, wt_ref, bb_ref,
                      m_ref, s_ref, q_ref, *, nrows, kk):
    mu, sc, bt = _norm_consts(si_ref, qi_ref, g_ref, bt_ref, nrows)
    x = jnp.maximum((y_ref[...] - mu) * sc + bt, 0.0)
    y = jnp.dot(x, wt_ref[...], preferred_element_type=_F32) + bb_ref[...]
    rg = y.shape[0] // kk
    m_ref[...] = jnp.max(y.reshape(rg, kk, y.shape[1]), axis=1)
    _accum_stats(y, s_ref, q_ref)


def _finalize_body(y_ref, si_ref, qi_ref, g_ref, bt_ref, o_ref, *, nrows):
    mu, sc, bt = _norm_consts(si_ref, qi_ref, g_ref, bt_ref, nrows)
    o_ref[...] = jnp.maximum((y_ref[...] - mu) * sc + bt, 0.0)


def _head_body(y_ref, si_ref, qi_ref, g_ref, bt_ref, wt_ref, bb_ref, o_ref,
               *, nrows):
    mu, sc, bt = _norm_consts(si_ref, qi_ref, g_ref, bt_ref, nrows)
    x = jnp.maximum((y_ref[...] - mu) * sc + bt, 0.0)
    z = jnp.dot(x, wt_ref[...], preferred_element_type=_F32) + bb_ref[...]
    o_ref[...] = jax.nn.sigmoid(z)


def _prep_w(w, cin_pad):
    cout, cin = w.shape
    wt = jnp.zeros((cin_pad, cout), _F32)
    return wt.at[:cin, :].set(w.T)


def _chunk_rows(rows, cmax):
    c = 8192 if cmax <= 256 else 4096
    return min(rows, c)


def _vec_spec(cout):
    return pl.BlockSpec((1, cout), lambda i: (0, 0))


def _lin_call(x, w, bias, *, group=None):
    rows, cin_pad = x.shape
    cout = w.shape[0]
    cr = _chunk_rows(rows, max(cin_pad, cout))
    grid = (rows // cr,)
    wt = _prep_w(w, cin_pad)
    bb = bias.reshape(1, cout)
    out_shape = [
        jax.ShapeDtypeStruct((rows, cout), _F32),
        jax.ShapeDtypeStruct((1, cout), _F32),
        jax.ShapeDtypeStruct((1, cout), _F32),
    ]
    out_specs = [
        pl.BlockSpec((cr, cout), lambda i: (i, 0)),
        _vec_spec(cout),
        _vec_spec(cout),
    ]
    if group is None:
        return pl.pallas_call(
            _lin_body,
            grid=grid,
            in_specs=[
                pl.BlockSpec((cr, cin_pad), lambda i: (i, 0)),
                pl.BlockSpec((cin_pad, cout), lambda i: (0, 0)),
                _vec_spec(cout),
            ],
            out_specs=out_specs,
            out_shape=out_shape,
        )(x, wt, bb)
    cx, cy, cz, kk, radius = group
    crg = cr // kk
    return pl.pallas_call(
        functools.partial(_group_lin_body, kk=kk, radius=radius),
        grid=grid,
        in_specs=[
            pl.BlockSpec((cr, cin_pad), lambda i: (i, 0)),
            pl.BlockSpec((crg, 1), lambda i: (i, 0)),
            pl.BlockSpec((crg, 1), lambda i: (i, 0)),
            pl.BlockSpec((crg, 1), lambda i: (i, 0)),
            pl.BlockSpec((cin_pad, cout), lambda i: (0, 0)),
            _vec_spec(cout),
        ],
        out_specs=out_specs,
        out_shape=out_shape,
    )(x, cx, cy, cz, wt, bb)


def _normlin_call(y, s, q, g, bt, w, bias):
    rows, cin = y.shape
    cout = w.shape[0]
    cr = _chunk_rows(rows, max(cin, cout))
    grid = (rows // cr,)
    wt = _prep_w(w, cin)
    bb = bias.reshape(1, cout)
    return pl.pallas_call(
        functools.partial(_normlin_body, nrows=rows),
        grid=grid,
        in_specs=[
            pl.BlockSpec((cr, cin), lambda i: (i, 0)),
            _vec_spec(cin),
            _vec_spec(cin),
            _vec_spec(cin),
            _vec_spec(cin),
            pl.BlockSpec((cin, cout), lambda i: (0, 0)),
            _vec_spec(cout),
        ],
        out_specs=[
            pl.BlockSpec((cr, cout), lambda i: (i, 0)),
            _vec_spec(cout),
            _vec_spec(cout),
        ],
        out_shape=[
            jax.ShapeDtypeStruct((rows, cout), _F32),
            jax.ShapeDtypeStruct((1, cout), _F32),
            jax.ShapeDtypeStruct((1, cout), _F32),
        ],
    )(y, s, q, g.reshape(1, cin), bt.reshape(1, cin), wt, bb)


def _normlin_max_call(y, s, q, g, bt, w, bias, kk):
    rows, cin = y.shape
    cout = w.shape[0]
    cr = _chunk_rows(rows, max(cin, cout))
    grid = (rows // cr,)
    wt = _prep_w(w, cin)
    bb = bias.reshape(1, cout)
    return pl.pallas_call(
        functools.partial(_normlin_max_body, nrows=rows, kk=kk),
        grid=grid,
        in_specs=[
            pl.BlockSpec((cr, cin), lambda i: (i, 0)),
            _vec_spec(cin),
            _vec_spec(cin),
            _vec_spec(cin),
            _vec_spec(cin),
            pl.BlockSpec((cin, cout), lambda i: (0, 0)),
            _vec_spec(cout),
        ],
        out_specs=[
            pl.BlockSpec((cr // kk, cout), lambda i: (i, 0)),
            _vec_spec(cout),
            _vec_spec(cout),
        ],
        out_shape=[
            jax.ShapeDtypeStruct((rows // kk, cout), _F32),
            jax.ShapeDtypeStruct((1, cout), _F32),
            jax.ShapeDtypeStruct((1, cout), _F32),
        ],
    )(y, s, q, g.reshape(1, cin), bt.reshape(1, cin), wt, bb)


def _finalize_call(y, s, q, g, bt, nrows=None):
    rows, cin = y.shape
    cr = _chunk_rows(rows, cin)
    grid = (rows // cr,)
    return pl.pallas_call(
        functools.partial(_finalize_body, nrows=nrows if nrows else rows),
        grid=grid,
        in_specs=[
            pl.BlockSpec((cr, cin), lambda i: (i, 0)),
            _vec_spec(cin),
            _vec_spec(cin),
            _vec_spec(cin),
            _vec_spec(cin),
        ],
        out_specs=pl.BlockSpec((cr, cin), lambda i: (i, 0)),
        out_shape=jax.ShapeDtypeStruct((rows, cin), _F32),
    )(y, s, q, g.reshape(1, cin), bt.reshape(1, cin))


def _head_call(y, s, q, g, bt, w2, b2):
    rows, cin = y.shape
    cout = w2.shape[0]
    cr = _chunk_rows(rows, cin)
    grid = (rows // cr,)
    wt = _prep_w(w2, cin)
    bb = b2.reshape(1, cout)
    return pl.pallas_call(
        functools.partial(_head_body, nrows=rows),
        grid=grid,
        in_specs=[
            pl.BlockSpec((cr, cin), lambda i: (i, 0)),
            _vec_spec(cin),
            _vec_spec(cin),
            _vec_spec(cin),
            _vec_spec(cin),
            pl.BlockSpec((cin, cout), lambda i: (0, 0)),
            _vec_spec(cout),
        ],
        out_specs=pl.BlockSpec((cr, cout), lambda i: (i, 0)),
        out_shape=jax.ShapeDtypeStruct((rows, cout), _F32),
    )(y, s, q, g.reshape(1, cin), bt.reshape(1, cin), wt, bb)


def _fp_body(x1_ref, y1_ref, z1_ref, x2_ref, y2_ref, z2_ref, p2_ref, o_ref,
             *, n1, n2):
    x1 = x1_ref[...].reshape(n1, 1)
    y1 = y1_ref[...].reshape(n1, 1)
    z1 = z1_ref[...].reshape(n1, 1)
    x2 = x2_ref[...].reshape(1, n2)
    y2 = y2_ref[...].reshape(1, n2)
    z2 = z2_ref[...].reshape(1, n2)
    dx = x1 - x2
    dy = y1 - y2
    dz = z1 - z2
    d = dx * dx + dy * dy + dz * dz
    iota = lax.broadcasted_iota(jnp.int32, (n1, n2), 1)
    picks = []
    for _ in range(3):
        mv = jnp.min(d, axis=1, keepdims=True)
        am = jnp.min(jnp.where(d <= mv, iota, n2), axis=1, keepdims=True)
        picks.append((1.0 / (mv + 1e-8), am))
        d = jnp.where(iota == am, jnp.inf, d)
    norm = picks[0][0] + picks[1][0] + picks[2][0]
    wm = jnp.zeros((n1, n2), _F32)
    for rec, am in picks:
        wm = wm + (iota == am).astype(_F32) * (rec / norm)
    o_ref[...] = jnp.dot(wm, p2_ref[0], preferred_element_type=_F32)


def _fp_interp(x1, y1, z1, x2, y2, z2, p2):
    b, n1 = x1.shape
    n2 = x2.shape[1]
    c2 = p2.shape[2]
    return pl.pallas_call(
        functools.partial(_fp_body, n1=n1, n2=n2),
        grid=(b,),
        in_specs=[
            pl.BlockSpec((1, 1, n1), lambda i: (i, 0, 0)),
            pl.BlockSpec((1, 1, n1), lambda i: (i, 0, 0)),
            pl.BlockSpec((1, 1, n1), lambda i: (i, 0, 0)),
            pl.BlockSpec((1, 1, n2), lambda i: (i, 0, 0)),
            pl.BlockSpec((1, 1, n2), lambda i: (i, 0, 0)),
            pl.BlockSpec((1, 1, n2), lambda i: (i, 0, 0)),
            pl.BlockSpec((1, n2, c2), lambda i: (i, 0, 0)),
        ],
        out_specs=pl.BlockSpec((n1, c2), lambda i: (i, 0)),
        out_shape=jax.ShapeDtypeStruct((b * n1, c2), _F32),
    )(x1.reshape(b, 1, n1), y1.reshape(b, 1, n1), z1.reshape(b, 1, n1),
      x2.reshape(b, 1, n2), y2.reshape(b, 1, n2), z2.reshape(b, 1, n2), p2)


def _pad_cols(a, cpad):
    rows, c = a.shape
    if c == cpad:
        return a
    return jnp.concatenate([a, jnp.zeros((rows, cpad - c), _F32)], axis=1)


def _set_abstraction(x, y, z, feats, npoint, radius, nsample, layers, cs):
    b, n = x.shape
    cfeat = feats.shape[2]
    cx, cy, cz = _fps(x, y, z, npoint)
    gidx = _ball_query(x, y, z, cx, cy, cz, radius, nsample, cs)
    cin = 3 + cfeat
    cpad = ((cin + 15) // 16) * 16
    table = jnp.concatenate(
        [x[..., None], y[..., None], z[..., None], feats], axis=2)
    table = _pad_cols(table.reshape(b * n, cin), cpad)
    w1, b1 = layers[0][0], layers[0][1]
    if cin <= 16:
        rows = _sc_gather(table, gidx.reshape(-1), cpad, False)
        cenx = cx.reshape(b * npoint, 1)
        ceny = cy.reshape(b * npoint, 1)
        cenz = cz.reshape(b * npoint, 1)
        yv, s, q = _lin_call(rows, w1, b1,
                             group=(cenx, ceny, cenz, nsample, radius))
    else:
        cout1 = w1.shape[0]
        wa = jnp.concatenate([w1[:, :3] / radius, w1[:, 3:]], axis=1)
        arows = _nostat_lin(table, wa, jnp.zeros((cout1,), _F32))
        dvec = jnp.concatenate(
            [cx.reshape(-1, 1), cy.reshape(-1, 1), cz.reshape(-1, 1),
             jnp.zeros((b * npoint, 5), _F32)], axis=1)
        wd = -w1[:, :3] / radius
        drows = _nostat_lin(dvec, wd, b1)
        yv, s, q = _onehot_group(gidx, arows.reshape(b, n, cout1), drows,
                                 nsample)
    g, bt = layers[0][2], layers[0][3]
    nrows_full = yv.shape[0]
    for (w, bias, g2, bt2) in layers[1:-1]:
        yv, s2, q2 = _normlin_call(yv, s, q, g, bt, w, bias)
        s, q, g, bt = s2, q2, g2, bt2
    w, bias, g2, bt2 = layers[-1]
    m, s2, q2 = _normlin_max_call(yv, s, q, g, bt, w, bias, nsample)
    pooled = _finalize_call(m, s2, q2, g2, bt2, nrows=nrows_full)
    return cx, cy, cz, pooled


def _feature_prop(x1, y1, z1, x2, y2, z2, p1flat, p2, layers, finalize):
    b, n1 = x1.shape
    interp = _fp_interp(x1, y1, z1, x2, y2, z2, p2)
    xin = jnp.concatenate([p1flat, interp], axis=1)
    yv, s, q = _lin_call(xin, layers[0][0], layers[0][1])
    g, bt = layers[0][2], layers[0][3]
    for (w, bias, g2, bt2) in layers[1:]:
        yv, s2, q2 = _normlin_call(yv, s, q, g, bt, w, bias)
        s, q, g, bt = s2, q2, g2, bt2
    if finalize:
        return _finalize_call(yv, s, q, g, bt)
    return yv, s, q, g, bt


def kernel(xyz, features, params):
    b, n, _ = xyz.shape
    x0 = xyz[..., 0]
    y0 = xyz[..., 1]
    z0 = xyz[..., 2]

    cx1, cy1, cz1, p1 = _set_abstraction(
        x0, y0, z0, features, 256, 0.05, 32, params['sa1'], cs=32)
    p1r = p1.reshape(b, 256, p1.shape[1])
    cx2, cy2, cz2, p2 = _set_abstraction(
        cx1, cy1, cz1, p1r, 128, 0.1, 64, params['sa2'], cs=128)
    p2r = p2.reshape(b, 128, p2.shape[1])
    cx3, cy3, cz3, p3 = _set_abstraction(
        cx2, cy2, cz2, p2r, 64, 0.2, 128, params['sa3'], cs=64)
    p3r = p3.reshape(b, 64, p3.shape[1])

    l2f = _feature_prop(cx2, cy2, cz2, cx3, cy3, cz3, p2, p3r,
                        params['fp3'], finalize=True)
    l2fr = l2f.reshape(b, 128, l2f.shape[1])
    l1f = _feature_prop(cx1, cy1, cz1, cx2, cy2, cz2, p1, l2fr,
                        params['fp2'], finalize=True)
    l1fr = l1f.reshape(b, 256, l1f.shape[1])
    f0flat = features.reshape(b * n, features.shape[2])
    yv, s, q, g, bt = _feature_prop(x0, y0, z0, cx1, cy1, cz1, f0flat, l1fr,
                                    params['fp1'], finalize=False)

    w1, b1, g1, bt1 = params['head1']
    yh, sh, qh = _normlin_call(yv, s, q, g, bt, w1, b1)
    w2, b2 = params['head2']
    out = _head_call(yh, sh, qh, g1, bt1, w2, b2)
    return out.reshape(b, n)
